# Initial kernel scaffold; baseline (speedup 1.0000x reference)
import jax
import jax.numpy as jnp
from jax.experimental import pallas as pl


def kernel(feats, labels):
    raise NotImplementedError("write your pallas kernel here")


# speedup vs baseline: 5.5221x; 5.5221x over previous
import functools

import jax
import jax.numpy as jnp
from jax.experimental import pallas as pl

TEMP_INV = 1.0 / 0.07
HARD_K = 10
ROW_BLOCK = 256


def _rotl(x, d):
    return (x << jnp.uint32(d)) | (x >> jnp.uint32(32 - d))


def _threefry_uniform(p):
    k1 = jnp.uint32(0)
    k2 = jnp.uint32(42)
    kx = k1 ^ k2 ^ jnp.uint32(0x1BD11BDA)
    ks = (k1, k2, kx)
    rot0 = (13, 15, 26, 6)
    rot1 = (17, 29, 16, 24)

    x0 = jnp.zeros_like(p, dtype=jnp.uint32) + ks[0]
    x1 = p.astype(jnp.uint32) + ks[1]
    for rots, a, b, inc in (
        (rot0, 1, 2, 1),
        (rot1, 2, 0, 2),
        (rot0, 0, 1, 3),
        (rot1, 1, 2, 4),
        (rot0, 2, 0, 5),
    ):
        for r in rots:
            x0 = x0 + x1
            x1 = _rotl(x1, r)
            x1 = x0 ^ x1
        x0 = x0 + ks[a]
        x1 = x1 + ks[b] + jnp.uint32(inc)
    bits = x0 ^ x1
    fbits = (bits >> jnp.uint32(9)) | jnp.uint32(0x3F800000)
    return jax.lax.bitcast_convert_type(fbits, jnp.float32) - 1.0


def _body(feats_ref, labrow_ref, labcol_ref, tot_ref, cnt_ref, *, B, R):
    i = pl.program_id(0)

    @pl.when(i == 0)
    def _init():
        tot_ref[:, :] = jnp.zeros((1, 1), jnp.float32)
        cnt_ref[:, :] = jnp.zeros((1, 1), jnp.float32)

    f = feats_ref[:, :]
    nrm = jnp.sqrt(jnp.sum(f * f, axis=1, keepdims=True))
    z = f / jnp.maximum(nrm, 1e-12)

    fr = feats_ref[pl.ds(i * R, R), :]
    nrm_r = jnp.sqrt(jnp.sum(fr * fr, axis=1, keepdims=True))
    zr = fr / jnp.maximum(nrm_r, 1e-12)

    sim = jax.lax.dot_general(
        zr, z, (((1,), (1,)), ((), ())),
        preferred_element_type=jnp.float32) * TEMP_INV

    lab_r = labrow_ref[:, :]
    lab_c = labcol_ref[:, :]
    col = jax.lax.broadcasted_iota(jnp.int32, (R, B), 1)
    row_g = i * R + jax.lax.broadcasted_iota(jnp.int32, (R, B), 0)

    eq = lab_r == lab_c
    same = eq & (col != row_g)
    diff = ~eq

    u = _threefry_uniform(row_g * B + col)
    u = jnp.where(same, u, -1.0)
    um = jnp.max(u, axis=1, keepdims=True)
    is_max = u == um
    pos_idx = jnp.min(jnp.where(is_max, col, B), axis=1, keepdims=True)
    pos_logit = jnp.sum(jnp.where(col == pos_idx, sim, 0.0), axis=1,
                        keepdims=True)

    neg = jnp.where(diff, sim, -1e30)
    s = jnp.zeros((R, 1), jnp.float32)
    budget = jnp.full((R, 1), float(HARD_K), jnp.float32)
    for _ in range(HARD_K):
        m = jnp.max(neg, axis=1, keepdims=True)
        hit = neg == m
        c = jnp.sum(hit.astype(jnp.float32), axis=1, keepdims=True)
        t = jnp.minimum(c, budget)
        s = s + t * jnp.exp(m)
        budget = budget - t
        neg = jnp.where(hit, -1e30, neg)

    num = jnp.exp(pos_logit)
    den = num + s
    loss = -jnp.log(jnp.clip(num / jnp.clip(den, 1e-8, None), 1e-8, None))

    any_same = um >= 0.0
    any_diff = jnp.sum(diff.astype(jnp.float32), axis=1, keepdims=True) > 0.0
    valid = any_same & any_diff

    tot_ref[:, :] += jnp.sum(jnp.where(valid, loss, 0.0), axis=0,
                             keepdims=True)
    cnt_ref[:, :] += jnp.sum(valid.astype(jnp.float32), axis=0,
                             keepdims=True)


def kernel(feats, labels):
    B, D = feats.shape
    R = ROW_BLOCK
    G = B // R
    labels_i = labels.astype(jnp.int32)
    lab_row = labels_i.reshape(B, 1)
    lab_col = labels_i.reshape(1, B)

    tot, cnt = pl.pallas_call(
        functools.partial(_body, B=B, R=R),
        grid=(G,),
        in_specs=[
            pl.BlockSpec((B, D), lambda i: (0, 0)),
            pl.BlockSpec((R, 1), lambda i: (i, 0)),
            pl.BlockSpec((1, B), lambda i: (0, 0)),
        ],
        out_specs=[
            pl.BlockSpec((1, 1), lambda i: (0, 0)),
            pl.BlockSpec((1, 1), lambda i: (0, 0)),
        ],
        out_shape=[
            jax.ShapeDtypeStruct((1, 1), jnp.float32),
            jax.ShapeDtypeStruct((1, 1), jnp.float32),
        ],
    )(feats, lab_row, lab_col)

    total = tot[0, 0]
    n_valid = cnt[0, 0]
    return jnp.where(n_valid > 0, total / jnp.maximum(n_valid, 1.0),
                     jnp.zeros(()))
